# baseline (speedup 1.0000x reference)
import functools

import jax
import jax.numpy as jnp
from jax import lax
from jax.experimental import pallas as pl
from jax.experimental.pallas import tpu as pltpu
from jax.experimental.pallas import tpu_sc as plsc

_FACTOR = 20.0
_ROWS = 32768
_COLS = 512
_N = _ROWS * _COLS

_TC_ROWS = 24576
_SC_ROWS = _ROWS - _TC_ROWS

_BLOCK_ROWS = 4096
_TC_GRID = _TC_ROWS // _BLOCK_ROWS

_NC, _NS, _L = 2, 16, 16
_NW = _NC * _NS
_RPW = _SC_ROWS // _NW
_CR = 32
_NCHUNK = _RPW // _CR


def _wmse_tc_body(x_ref, t_ref, out_ref):
    i = pl.program_id(0)
    x = x_ref[...]
    t = t_ref[...]
    d = x - t
    sq = d * d
    fl = t > 0.0
    s_all = jnp.sum(sq)
    s_fl = jnp.sum(jnp.where(fl, sq, 0.0))
    c_fl = jnp.sum(jnp.where(fl, 1.0, 0.0))

    @pl.when(i == 0)
    def _init():
        out_ref[0] = 0.0
        out_ref[1] = 0.0
        out_ref[2] = 0.0

    out_ref[0] += s_all
    out_ref[1] += s_fl
    out_ref[2] += c_fl


def _tree_sum(vals):
    vals = list(vals)
    while len(vals) > 1:
        nxt = [vals[i] + vals[i + 1] for i in range(0, len(vals) - 1, 2)]
        if len(vals) % 2:
            nxt.append(vals[-1])
        vals = nxt
    return vals[0]


_mesh = plsc.VectorSubcoreMesh(core_axis_name="c", subcore_axis_name="s")


@functools.partial(
    pl.kernel,
    mesh=_mesh,
    out_type=jax.ShapeDtypeStruct((_NW * 3 * _L,), jnp.float32),
    scratch_types=[
        pltpu.VMEM((_CR, _COLS), jnp.float32),
        pltpu.VMEM((_CR, _COLS), jnp.float32),
        pltpu.VMEM((_CR, _COLS), jnp.float32),
        pltpu.VMEM((_CR, _COLS), jnp.float32),
        pltpu.VMEM((3 * _L,), jnp.float32),
        pltpu.SemaphoreType.DMA,
        pltpu.SemaphoreType.DMA,
        pltpu.SemaphoreType.DMA,
        pltpu.SemaphoreType.DMA,
    ],
    compiler_params=pltpu.CompilerParams(use_tc_tiling_on_sc=True),
)
def _wmse_sc(x_hbm, t_hbm, out_hbm, xa, ta, xb, tb, accv, sxa, sta, sxb, stb):
    wid = lax.axis_index("s") * _NC + lax.axis_index("c")
    base = _TC_ROWS + wid * _RPW

    def issue(chunk, xbuf, tbuf, sx, st):
        row0 = base + chunk * _CR
        pltpu.async_copy(x_hbm.at[pl.ds(row0, _CR)], xbuf, sx)
        pltpu.async_copy(t_hbm.at[pl.ds(row0, _CR)], tbuf, st)

    def drain(xbuf, tbuf, sx, st):
        pltpu.make_async_copy(x_hbm.at[pl.ds(0, _CR)], xbuf, sx).wait()
        pltpu.make_async_copy(t_hbm.at[pl.ds(0, _CR)], tbuf, st).wait()

    def compute(xbuf, tbuf, acc):
        def qbody(q, carry):
            a_all, a_fl, a_c = carry
            r = q >> 2
            c0 = (q & 3) << 7
            alls, fls, cs = [], [], []
            for k in range(8):
                s0 = pl.multiple_of(c0 + k * _L, _L)
                x = xbuf[r, pl.ds(s0, _L)]
                t = tbuf[r, pl.ds(s0, _L)]
                d = x - t
                sq = d * d
                m = t > 0.0
                alls.append(sq)
                fls.append(jnp.where(m, sq, 0.0))
                cs.append(jnp.where(m, 1.0, 0.0))
            return (a_all + _tree_sum(alls), a_fl + _tree_sum(fls),
                    a_c + _tree_sum(cs))

        return lax.fori_loop(0, _CR * 4, qbody, acc)

    issue(0, xa, ta, sxa, sta)
    issue(1, xb, tb, sxb, stb)

    zero = jnp.zeros((_L,), jnp.float32)
    acc = (zero, zero, zero)

    def pair_body(j, carry):
        i0 = 2 * j
        drain(xa, ta, sxa, sta)
        carry = compute(xa, ta, carry)
        issue(i0 + 2, xa, ta, sxa, sta)
        drain(xb, tb, sxb, stb)
        carry = compute(xb, tb, carry)
        issue(i0 + 3, xb, tb, sxb, stb)
        return carry

    acc = lax.fori_loop(0, _NCHUNK // 2 - 1, pair_body, acc)
    drain(xa, ta, sxa, sta)
    acc = compute(xa, ta, acc)
    drain(xb, tb, sxb, stb)
    acc = compute(xb, tb, acc)

    accv[pl.ds(0, _L)] = acc[0]
    accv[pl.ds(_L, _L)] = acc[1]
    accv[pl.ds(2 * _L, _L)] = acc[2]
    pltpu.sync_copy(accv, out_hbm.at[pl.ds(wid * 3 * _L, 3 * _L)])


def _finalize(sums, n):
    s_all, s_fl, c_fl = sums[0], sums[1], sums[2]
    s_un = s_all - s_fl
    c_un = n - c_fl
    flood_loss = jnp.where(c_fl > 0, s_fl / jnp.maximum(c_fl, 1.0), 0.0)
    unflood_loss = jnp.where(c_un > 0, s_un / jnp.maximum(c_un, 1.0), 0.0)
    loss = _FACTOR * flood_loss + unflood_loss
    return (loss, flood_loss, unflood_loss)


@jax.jit
def kernel(inputs, targets):
    x2 = inputs.reshape(_ROWS, _COLS)
    t2 = targets.reshape(_ROWS, _COLS)
    sc_partials = _wmse_sc(x2, t2)
    tc_sums = pl.pallas_call(
        _wmse_tc_body,
        grid=(_TC_GRID,),
        in_specs=[
            pl.BlockSpec((_BLOCK_ROWS, _COLS), lambda i: (i, 0)),
            pl.BlockSpec((_BLOCK_ROWS, _COLS), lambda i: (i, 0)),
        ],
        out_specs=pl.BlockSpec(memory_space=pltpu.SMEM),
        out_shape=jax.ShapeDtypeStruct((3,), jnp.float32),
    )(x2, t2)
    sc_sums = sc_partials.reshape(_NW, 3, _L).sum(axis=(0, 2))
    return _finalize(tc_sums + sc_sums, jnp.float32(_N))


# speedup vs baseline: 1.3379x; 1.3379x over previous
import jax
import jax.numpy as jnp
from jax.experimental import pallas as pl
from jax.experimental.pallas import tpu as pltpu

_FACTOR = 20.0
_ROWS = 32768
_COLS = 512
_BLOCK_ROWS = 4096
_GRID = _ROWS // _BLOCK_ROWS


def _wmse_body(x_ref, t_ref, out_ref):
    i = pl.program_id(0)
    x = x_ref[...]
    t = t_ref[...]
    d = x - t
    sq = d * d
    fl = t > 0.0
    s_fl = jnp.sum(jnp.where(fl, sq, 0.0))
    s_un = jnp.sum(jnp.where(fl, 0.0, sq))
    c_fl = jnp.sum(jnp.where(fl, 1.0, 0.0))

    @pl.when(i == 0)
    def _init():
        out_ref[0] = 0.0
        out_ref[1] = 0.0
        out_ref[2] = 0.0

    out_ref[0] += s_fl
    out_ref[1] += s_un
    out_ref[2] += c_fl


def _finalize(sums, n):
    s_fl, s_un, c_fl = sums[0], sums[1], sums[2]
    c_un = n - c_fl
    flood_loss = jnp.where(c_fl > 0, s_fl / jnp.maximum(c_fl, 1.0), 0.0)
    unflood_loss = jnp.where(c_un > 0, s_un / jnp.maximum(c_un, 1.0), 0.0)
    loss = _FACTOR * flood_loss + unflood_loss
    return (loss, flood_loss, unflood_loss)


@jax.jit
def kernel(inputs, targets):
    n = inputs.size
    x = inputs.reshape(_ROWS, _COLS)
    t = targets.reshape(_ROWS, _COLS)
    sums = pl.pallas_call(
        _wmse_body,
        grid=(_GRID,),
        in_specs=[
            pl.BlockSpec((_BLOCK_ROWS, _COLS), lambda i: (i, 0)),
            pl.BlockSpec((_BLOCK_ROWS, _COLS), lambda i: (i, 0)),
        ],
        out_specs=pl.BlockSpec(memory_space=pltpu.SMEM),
        out_shape=jax.ShapeDtypeStruct((3,), jnp.float32),
    )(x, t)
    return _finalize(sums, jnp.float32(n))
